# speedup vs baseline: 2.3434x; 2.3434x over previous
import functools

import jax
import jax.numpy as jnp
from jax import lax
from jax.experimental import pallas as pl
from jax.experimental.pallas import tpu as pltpu
from jax.experimental.pallas import tpu_sc as plsc

_BATCH = 16384
_SEQ = 10
_DIM = 128
_NUM_ROWS = _BATCH * _SEQ

_NC = 2
_NS = 16
_NW = _NC * _NS
_ROWS_PER_W = _NUM_ROWS // _NW
_CHUNK = 128
_NJ = _ROWS_PER_W // _CHUNK
_NBUF = 2


@functools.partial(
    pl.kernel,
    mesh=plsc.VectorSubcoreMesh(core_axis_name="c", subcore_axis_name="s"),
    out_type=jax.ShapeDtypeStruct((_NUM_ROWS, _DIM), jnp.float32),
    scratch_types=[
        pltpu.VMEM((_NJ, _CHUNK), jnp.int32),
        pltpu.VMEM((_CHUNK, _DIM), jnp.float32),
        pltpu.VMEM((_CHUNK, _DIM), jnp.float32),
        pltpu.SemaphoreType.DMA,
        pltpu.SemaphoreType.DMA,
    ],
)
def _gather_rows(idx_hbm, table_hbm, out_hbm, idx_v, buf0, buf1, sem0, sem1):
    wid = lax.axis_index("s") * _NC + lax.axis_index("c")
    base = wid * _ROWS_PER_W
    pltpu.sync_copy(idx_hbm.at[pl.ds(wid * _NJ, _NJ)], idx_v)
    bufs = (buf0, buf1)
    sems = (sem0, sem1)
    for b in range(_NBUF):
        pltpu.async_copy(table_hbm.at[idx_v.at[b]], bufs[b], sems[b])

    def body(g, carry):
        for b in range(_NBUF):
            j = g * _NBUF + b
            buf, sem = bufs[b], sems[b]
            pltpu.make_async_copy(table_hbm.at[idx_v.at[j]], buf, sem).wait()
            pltpu.sync_copy(buf, out_hbm.at[pl.ds(base + j * _CHUNK, _CHUNK)])

            @pl.when(j + _NBUF < _NJ)
            def _():
                pltpu.async_copy(table_hbm.at[idx_v.at[j + _NBUF]], buf, sem)

        return carry

    lax.fori_loop(0, _NJ // _NBUF, body, 0)


def kernel(x, table):
    idx = x.reshape(-1).astype(jnp.int32).reshape(_NW * _NJ, _CHUNK)
    rows = _gather_rows(idx, table)
    return rows.reshape(_BATCH, _SEQ * _DIM)


# speedup vs baseline: 2.3757x; 1.0138x over previous
import functools

import jax
import jax.numpy as jnp
from jax import lax
from jax.experimental import pallas as pl
from jax.experimental.pallas import tpu as pltpu
from jax.experimental.pallas import tpu_sc as plsc

_BATCH = 16384
_SEQ = 10
_DIM = 128
_NUM_ROWS = _BATCH * _SEQ

_NC = 2
_NS = 16
_NW = _NC * _NS
_ROWS_PER_W = _NUM_ROWS // _NW
_CHUNK = 128
_NJ = _ROWS_PER_W // _CHUNK
_NBUF = 5
_DEPTH = 3


@functools.partial(
    pl.kernel,
    mesh=plsc.VectorSubcoreMesh(core_axis_name="c", subcore_axis_name="s"),
    out_type=jax.ShapeDtypeStruct((_NUM_ROWS, _DIM), jnp.float32),
    scratch_types=[
        pltpu.VMEM((_NJ, _CHUNK), jnp.int32),
        [pltpu.VMEM((_CHUNK, _DIM), jnp.float32) for _ in range(_NBUF)],
        [pltpu.SemaphoreType.DMA for _ in range(_NBUF)],
        [pltpu.SemaphoreType.DMA for _ in range(_NBUF)],
    ],
)
def _gather_rows(idx_hbm, table_hbm, out_hbm, idx_v, bufs, gsems, ssems):
    wid = lax.axis_index("s") * _NC + lax.axis_index("c")
    base = wid * _ROWS_PER_W
    pltpu.sync_copy(idx_hbm.at[pl.ds(wid * _NJ, _NJ)], idx_v)

    def gather(j, b):
        return pltpu.make_async_copy(table_hbm.at[idx_v.at[j]], bufs[b], gsems[b])

    def scatter(j, b):
        return pltpu.make_async_copy(
            bufs[b], out_hbm.at[pl.ds(base + j * _CHUNK, _CHUNK)], ssems[b]
        )

    for b in range(_DEPTH):
        gather(b, b).start()

    def body(g, carry):
        for b in range(_NBUF):
            j = g * _NBUF + b
            gather(j, b).wait()
            scatter(j, b).start()
            b2 = (b + _DEPTH) % _NBUF

            @pl.when(j + _DEPTH < _NJ)
            def _():
                @pl.when(j >= _NBUF - _DEPTH)
                def _():
                    scatter(j + _DEPTH - _NBUF, b2).wait()

                gather(j + _DEPTH, b2).start()

        return carry

    lax.fori_loop(0, _NJ // _NBUF, body, 0)
    for b in range(_NBUF):
        scatter(_NJ - _NBUF + b, (_NJ - _NBUF + b) % _NBUF).wait()


def kernel(x, table):
    idx = x.reshape(-1).astype(jnp.int32).reshape(_NW * _NJ, _CHUNK)
    rows = _gather_rows(idx, table)
    return rows.reshape(_BATCH, _SEQ * _DIM)


# speedup vs baseline: 4.7056x; 1.9807x over previous
import functools

import jax
import jax.numpy as jnp
from jax import lax
from jax.experimental import pallas as pl
from jax.experimental.pallas import tpu as pltpu
from jax.experimental.pallas import tpu_sc as plsc

_BATCH = 16384
_SEQ = 10
_DIM = 128

_NC = 2
_NS = 16
_NW = _NC * _NS
_GRP = 8
_GROW = _GRP * _SEQ
_NGRP = _BATCH // _GRP
_NJ = _NGRP // _NW
_NBUF = 8
_DEPTH = 4


@functools.partial(
    pl.kernel,
    mesh=plsc.VectorSubcoreMesh(core_axis_name="c", subcore_axis_name="s"),
    out_type=jax.ShapeDtypeStruct((_BATCH, _SEQ * _DIM), jnp.float32),
    scratch_types=[
        pltpu.VMEM((_NJ, _GROW), jnp.int32),
        [pltpu.VMEM((_GROW, _DIM), jnp.float32) for _ in range(_NBUF)],
        [pltpu.SemaphoreType.DMA for _ in range(_NBUF)],
        [pltpu.SemaphoreType.DMA for _ in range(_NBUF)],
    ],
)
def _gather_rows(idx_hbm, table_hbm, out_hbm, idx_v, bufs, gsems, ssems):
    wid = lax.axis_index("s") * _NC + lax.axis_index("c")
    base = wid * _NJ
    pltpu.sync_copy(idx_hbm.at[pl.ds(base, _NJ)], idx_v)

    def gather(j, b):
        return pltpu.make_async_copy(table_hbm.at[idx_v.at[j]], bufs[b], gsems[b])

    def scatter(j, b):
        return pltpu.make_async_copy(
            bufs[b].reshape(_GRP, _SEQ * _DIM),
            out_hbm.at[pl.ds((base + j) * _GRP, _GRP)],
            ssems[b],
        )

    for b in range(_DEPTH):
        gather(b, b).start()

    def body(g, carry):
        for b in range(_NBUF):
            j = g * _NBUF + b
            gather(j, b).wait()
            scatter(j, b).start()
            b2 = (b + _DEPTH) % _NBUF

            @pl.when(j + _DEPTH < _NJ)
            def _():
                @pl.when(j >= _NBUF - _DEPTH)
                def _():
                    scatter(j + _DEPTH - _NBUF, b2).wait()

                gather(j + _DEPTH, b2).start()

        return carry

    lax.fori_loop(0, _NJ // _NBUF, body, 0)
    for b in range(_NBUF):
        scatter(_NJ - _NBUF + b, (_NJ - _NBUF + b) % _NBUF).wait()


def kernel(x, table):
    idx = x.astype(jnp.int32).reshape(_NGRP, _GROW)
    return _gather_rows(idx, table)
